# Initial kernel scaffold; baseline (speedup 1.0000x reference)
import jax
import jax.numpy as jnp
from jax.experimental import pallas as pl


def kernel(input, adj, A_tilde, A_tilde_val, adj_sct_o1, adj_sct_o1_val, adj_sct_o2, adj_sct_o2_val, weight0, weight1, weight2, weight3, weight4, bias0, bias1, bias2, bias3, bias4):
    raise NotImplementedError("write your pallas kernel here")


# speedup vs baseline: 10.2883x; 10.2883x over previous
import functools

import jax
import jax.numpy as jnp
from jax import lax
from jax.experimental import pallas as pl
from jax.experimental.pallas import tpu as pltpu
from jax.experimental.pallas import tpu_sc as plsc

N = 10000
E = 320000
NCORE = 2
NSUB = 16
LANES = 16

IB = 80
BLK_ROWS = 8
KB = IB * BLK_ROWS
UNITS = E // KB
WB_ROWS = 2000


def _make_spmm(wh):
    mesh = plsc.VectorSubcoreMesh(core_axis_name="c", subcore_axis_name="s")

    @functools.partial(
        pl.kernel,
        out_type=jax.ShapeDtypeStruct((NCORE * N, wh), jnp.float32),
        mesh=mesh,
        scratch_types=[
            pltpu.VMEM((BLK_ROWS, IB), jnp.int32),
            pltpu.VMEM((BLK_ROWS, IB), jnp.int32),
            pltpu.VMEM((KB,), jnp.float32),
            pltpu.VMEM((KB, wh), jnp.float32),
            pltpu.VMEM_SHARED((N, wh), jnp.float32),
            pltpu.SemaphoreType.DMA,
        ],
        compiler_params=pltpu.CompilerParams(use_tc_tiling_on_sc=False),
    )
    def spmm(x_hbm, src_hbm, dst_hbm, val_hbm, out_hbm,
             src_v, dst_v, val_v, rows_v, acc, sem):
        c = lax.axis_index("c")
        s = lax.axis_index("s")
        coff = c * N

        def zero_row(i, carry):
            for t in range(wh // LANES):
                rows_v[i, pl.ds(t * LANES, LANES)] = jnp.zeros((LANES,), jnp.float32)
            return carry
        lax.fori_loop(0, IB, zero_row, 0)

        @pl.when(s < N // WB_ROWS)
        def _():
            for k in range(WB_ROWS // IB):
                pltpu.sync_copy(rows_v.at[pl.ds(0, IB)],
                                acc.at[pl.ds(s * WB_ROWS + k * IB, IB)])
        plsc.subcore_barrier()

        nblk = jnp.where(s < 4, 32, 31)
        ustart = s * 31 + jnp.minimum(s, 4)

        def block(b, carry):
            u = ustart + b
            rbase = u * BLK_ROWS
            pltpu.sync_copy(src_hbm.at[pl.ds(rbase, BLK_ROWS)], src_v)
            pltpu.sync_copy(dst_hbm.at[pl.ds(rbase, BLK_ROWS)], dst_v)
            pltpu.sync_copy(val_hbm.at[pl.ds(u * KB, KB)], val_v)
            for j in range(BLK_ROWS):
                for t in range(IB // LANES):
                    sl = pl.ds(t * LANES, LANES)
                    src_v[j, sl] = src_v[j, sl] + coff
            cps = [
                pltpu.async_copy(x_hbm.at[src_v.at[j]],
                                 rows_v.at[pl.ds(j * IB, IB)], sem)
                for j in range(BLK_ROWS)
            ]
            for cp in cps:
                cp.wait()

            def scale(q, carry2):
                v16 = val_v[pl.ds(q * LANES, LANES)]
                for l in range(LANES):
                    vl = v16[l]
                    e = q * LANES + l
                    for t in range(wh // LANES):
                        sl = pl.ds(t * LANES, LANES)
                        rows_v[e, sl] = rows_v[e, sl] * vl
                return carry2
            lax.fori_loop(0, KB // LANES, scale, 0)

            for j in range(BLK_ROWS):
                pltpu.sync_copy(rows_v.at[pl.ds(j * IB, IB)],
                                acc.at[dst_v.at[j]], add=True)
            return carry
        lax.fori_loop(0, nblk, block, 0)

        plsc.subcore_barrier()

        @pl.when(s < N // WB_ROWS)
        def _():
            pltpu.sync_copy(acc.at[pl.ds(s * WB_ROWS, WB_ROWS)],
                            out_hbm.at[pl.ds(coff + s * WB_ROWS, WB_ROWS)])

    return spmm


_spmm48 = _make_spmm(48)
_spmm32 = _make_spmm(32)
_spmm16 = _make_spmm(16)


def _matmul(x, w):
    bn = 1000

    def mm(x_ref, w_ref, o_ref):
        o_ref[...] = jnp.dot(x_ref[...], w_ref[...],
                             preferred_element_type=jnp.float32)

    return pl.pallas_call(
        mm,
        grid=(N // bn,),
        in_specs=[
            pl.BlockSpec((bn, 128), lambda i: (i, 0)),
            pl.BlockSpec((128, 160), lambda i: (0, 0)),
        ],
        out_specs=pl.BlockSpec((bn, 160), lambda i: (i, 0)),
        out_shape=jax.ShapeDtypeStruct((N, 160), jnp.float32),
    )(x, w)


def _split_cols(m):
    g = m.shape[1] // 32
    m4 = m.reshape(N, g, 2, 16)
    return m4.transpose(2, 0, 1, 3).reshape(2 * N, g * 16)


def _unsplit(y):
    g = y.shape[1] // 16
    return y.reshape(2, N, g, 16).transpose(1, 2, 0, 3).reshape(N, g * 32)


def kernel(input, adj, A_tilde, A_tilde_val, adj_sct_o1, adj_sct_o1_val,
           adj_sct_o2, adj_sct_o2_val, weight0, weight1, weight2, weight3,
           weight4, bias0, bias1, bias2, bias3, bias4):
    w = jnp.concatenate([weight0, weight1, weight2, weight3, weight4], axis=1)
    sup = _matmul(input, w)

    src_a = A_tilde[1].reshape(E // IB, IB)
    dst_a = A_tilde[0].reshape(E // IB, IB)
    src_1 = adj_sct_o1[1].reshape(E // IB, IB)
    dst_1 = adj_sct_o1[0].reshape(E // IB, IB)
    src_2 = adj_sct_o2[1].reshape(E // IB, IB)
    dst_2 = adj_sct_o2[0].reshape(E // IB, IB)

    t1 = _split_cols(sup[:, 0:96])
    h1 = _spmm48(t1, src_a, dst_a, A_tilde_val)
    h1 = h1.reshape(2, N, 48)
    t2 = h1[:, :, 16:48].reshape(2 * N, 32)
    h2 = _spmm32(t2, src_a, dst_a, A_tilde_val)
    h2 = h2.reshape(2, N, 32)
    t3 = h2[:, :, 16:32].reshape(2 * N, 16)
    h3 = _spmm16(t3, src_a, dst_a, A_tilde_val)

    t4 = _split_cols(sup[:, 96:128])
    h4 = _spmm16(t4, src_1, dst_1, adj_sct_o1_val)
    t5 = _split_cols(sup[:, 128:160])
    h5 = _spmm16(t5, src_2, dst_2, adj_sct_o2_val)

    out0 = _unsplit(h1[:, :, 0:16].reshape(2 * N, 16)) + bias0
    out1 = _unsplit(h2[:, :, 0:16].reshape(2 * N, 16)) + bias1
    out2 = _unsplit(h3) + bias2
    out3 = _unsplit(h4) + bias3
    out4 = _unsplit(h5) + bias4
    return jnp.concatenate((out0, out1, out2, out3, out4), axis=1)
